# baseline (speedup 1.0000x reference)
import functools

import numpy as np
import jax
import jax.numpy as jnp
from jax import lax
from jax.experimental import pallas as pl
from jax.experimental.pallas import tpu as pltpu
from jax.experimental.pallas import tpu_sc as plsc

_B, _P, _K = 16, 1024, 30
_HI = jax.lax.Precision.HIGHEST

_NROWS = _B * _P
_NW = 32
_RPW = _NROWS // _NW
_BATCH = 4
_NB = _RPW // _BATCH


def _dot(a, w):
    return lax.dot_general(a, w, (((1,), (0,)), ((), ())), precision=_HI)


def _dist_and_tables(x, W1, b1):
    d = x.shape[1]
    d2 = jnp.sum(x * x, axis=1)
    G = lax.dot_general(x, x, (((1,), (1,)), ((), ())), precision=_HI)
    D = d2[:, None] + d2[None, :] - 2.0 * G
    W1a, W1b = W1[:d], W1[d:]
    A = _dot(x, W1a - W1b) + b1
    Bm = _dot(x, W1b)
    Bm = jnp.concatenate([Bm, jnp.zeros((_P, 64), jnp.float32)], axis=1)
    return D, A, Bm


def _edge_reduce(G, A, W2, b2):
    npt = A.shape[0]
    G = G[:, :64]
    h1 = jnp.maximum(A[:, None, :] + G.reshape(npt, _K, 64), 0.0)
    h2 = _dot(h1.reshape(npt * _K, 64), W2) + b2
    h2 = jnp.maximum(h2, 0.0).reshape(npt, _K, 64)
    return jnp.max(h2, axis=1)


def _tc_dist_body(x_ref, W1, b1, d_ref, a_ref, bm_ref):
    D, A, Bm = _dist_and_tables(x_ref[0], W1[...], b1[...])
    d_ref[0] = D
    a_ref[0] = A
    bm_ref[0] = Bm


def _tc_er_body(g_ref, a_ref, W2, b2, x_ref):
    x_ref[0] = _edge_reduce(g_ref[0], a_ref[0], W2[...], b2[...])


def _tc_cls_body(x1_ref, x2_ref, x3_ref, Wl1, bl1, Wm1, bm1,
                 Wm2, bm2, Wm3, bm3, out_ref):
    feat = jnp.concatenate([x1_ref[0], x2_ref[0], x3_ref[0]], axis=1)
    h = jnp.maximum(_dot(feat, Wl1[...]) + bl1[...], 0.0)
    h = jnp.maximum(_dot(h, Wm1[...]) + bm1[...], 0.0)
    h = jnp.maximum(_dot(h, Wm2[...]) + bm2[...], 0.0)
    h = _dot(h, Wm3[...]) + bm3[...]
    m = jnp.max(h, axis=1, keepdims=True)
    s = h - m
    lse = jnp.log(jnp.sum(jnp.exp(s), axis=1, keepdims=True))
    out_ref[0] = s - lse


def _wspec(a):
    nd = a.ndim
    return pl.BlockSpec(a.shape, lambda b, _n=nd: (0,) * _n)


def _bspec(shape):
    nz = len(shape)
    return pl.BlockSpec((1,) + shape, lambda b, _n=nz: (b,) + (0,) * _n)


_PT = 128
_NT = _P // _PT


def _tc_dist(x, W1, b1):
    d = x.shape[2]
    return pl.pallas_call(
        _tc_dist_body,
        grid=(_B,),
        in_specs=[_bspec((_P, d)), _wspec(W1), _wspec(b1)],
        out_specs=[_bspec((_P, _P)), _bspec((_P, 64)), _bspec((_P, 128))],
        out_shape=[jax.ShapeDtypeStruct((_B, _P, _P), jnp.float32),
                   jax.ShapeDtypeStruct((_B, _P, 64), jnp.float32),
                   jax.ShapeDtypeStruct((_B, _P, 128), jnp.float32)],
    )(x, W1, b1)


def _tspec(shape):
    nz = len(shape) - 1
    return pl.BlockSpec((1,) + shape,
                        lambda b, t, _n=nz: (b, t) + (0,) * _n)


def _tc_er(G, A, W2, b2):
    return pl.pallas_call(
        _tc_er_body,
        grid=(_B, _NT),
        in_specs=[_tspec((_PT * _K, 128)), _tspec((_PT, 64)),
                  pl.BlockSpec(W2.shape, lambda b, t: (0, 0)),
                  pl.BlockSpec(b2.shape, lambda b, t: (0,))],
        out_specs=_tspec((_PT, 64)),
        out_shape=jax.ShapeDtypeStruct((_B, _P, 64), jnp.float32),
    )(G, A, W2, b2)


def _tc_cls(x1, x2, x3, Wl1, bl1, Wm1, bm1, Wm2, bm2, Wm3, bm3):
    ws = [Wl1, bl1, Wm1, bm1, Wm2, bm2, Wm3, bm3]
    return pl.pallas_call(
        _tc_cls_body,
        grid=(_B,),
        in_specs=[_bspec((_P, 64))] * 3 + [_wspec(a) for a in ws],
        out_specs=_bspec((_P, 40)),
        out_shape=jax.ShapeDtypeStruct((_B, _P, 40), jnp.float32),
    )(x1, x2, x3, *ws)


def _merge32(L, H, s):
    rs = lax.rev(s, (0,))
    lo = jnp.minimum(L, rs)
    hi = jnp.maximum(L, rs)
    L2 = lax.sort(lo)
    hi_s = lax.sort(hi)
    lo2 = jnp.minimum(hi_s, lax.rev(H, (0,)))
    H2 = lax.sort(lo2)
    return L2, H2


def _lane(vec, j, iota, neutral):
    return jnp.max(jnp.where(iota == j, vec, neutral))


def _sc_body(d_hbm, bm_hbm, g_hbm, drow, sidx, pidx, eqbuf, fidx, gstage,
             gsem):
    wid = lax.axis_index("s") * 2 + lax.axis_index("c")
    base = wid * _RPW
    iota = lax.iota(jnp.int32, 16)
    inf16 = jnp.full((16,), np.inf, jnp.float32)
    ninf = jnp.float32(-np.inf)
    imax16 = jnp.full((16,), np.int32(2147483647), jnp.int32)

    for i in range(8):
        fidx[pl.ds(i * 16, 16)] = jnp.zeros((16,), jnp.int32)

    def select_row(p, pt0, cbase):
        pfull = jnp.full((16,), p, jnp.int32)
        m1 = drow[p, pl.ds(0, 16)]
        m2 = drow[p, pl.ds(16, 16)]
        for c in range(1, 32):
            m1 = jnp.minimum(m1, drow[p, pl.ds(32 * c, 16)])
            m2 = jnp.minimum(m2, drow[p, pl.ds(32 * c + 16, 16)])
        s1 = lax.sort(m1)
        s2 = lax.sort(m2)
        hi = jnp.maximum(s1, lax.rev(s2, (0,)))
        hi_s = lax.sort(hi)
        tb = jnp.full((16,), _lane(hi_s, 13, iota, ninf), jnp.float32)

        def f2(c, off):
            d = drow[p, pl.ds(c * 16, 16)]
            mask = d <= tb
            plsc.store_compressed(sidx.at[pl.ds(off, 16)], iota + c * 16,
                                  mask=mask)
            return off + jnp.sum(mask.astype(jnp.int32))

        n = lax.fori_loop(0, 64, f2, jnp.int32(0))
        nv = (n + 15) // 16

        def f3(v, LH):
            b16 = v * 16
            valid = (iota + b16) < n
            idxv = jnp.where(valid, sidx[pl.ds(b16, 16)], 0)
            keys = plsc.load_gather(drow, [pfull, idxv])
            keys = jnp.where(valid, keys, inf16)
            return _merge32(*LH, lax.sort(keys))

        _, H = lax.fori_loop(0, nv, f3, (inf16, inf16))
        t30 = jnp.full((16,), _lane(H, 13, iota, ninf), jnp.float32)

        def f4(v, carry):
            clt, neq = carry
            b16 = v * 16
            valid = (iota + b16) < n
            idxv = jnp.where(valid, sidx[pl.ds(b16, 16)], 0)
            keys = plsc.load_gather(drow, [pfull, idxv])
            klt = (keys < t30) & valid
            keq = (keys == t30) & valid
            plsc.store_compressed(pidx.at[pl.ds(clt, 16)], idxv, mask=klt)
            plsc.store_compressed(eqbuf.at[pl.ds(neq, 16)], idxv, mask=keq)
            return (clt + jnp.sum(klt.astype(jnp.int32)),
                    neq + jnp.sum(keq.astype(jnp.int32)))

        clt, neq = lax.fori_loop(0, nv, f4, (jnp.int32(0), jnp.int32(0)))
        need = 30 - clt

        nve = (neq + 15) // 16

        def f5(v, LH):
            b16 = v * 16
            valid = (iota + b16) < neq
            idxv = jnp.where(valid, eqbuf[pl.ds(b16, 16)], 0)
            idxv = jnp.where(valid, idxv, imax16)
            return _merge32(*LH, lax.sort(idxv))

        Li, Hi = lax.fori_loop(0, nve, f5, (imax16, imax16))
        nlo = jnp.minimum(need, 16)
        plsc.store_compressed(pidx.at[pl.ds(clt, 16)], Li, mask=iota < nlo)
        plsc.store_compressed(pidx.at[pl.ds(clt + nlo, 16)], Hi,
                              mask=iota < (need - 16))

        fb = p * 30
        fidx[pl.ds(fb, 16)] = pidx[pl.ds(0, 16)] + cbase
        plsc.store_compressed(fidx.at[pl.ds(fb + 16, 16)],
                              pidx[pl.ds(16, 16)] + cbase, mask=iota < 14)

    def batch_body(bi, _):
        pt0 = base + bi * _BATCH
        cbase = (pt0 // _P) * _P
        pltpu.sync_copy(d_hbm.at[pl.ds(pt0, _BATCH)], drow)
        for p in range(_BATCH):
            select_row(p, pt0, cbase)
        pltpu.async_copy(bm_hbm.at[fidx], gstage, gsem).wait()
        pltpu.sync_copy(gstage.at[pl.ds(0, _BATCH * 30)],
                        g_hbm.at[pl.ds(pt0 * 30, _BATCH * 30)])
        return 0

    lax.fori_loop(0, _NB, batch_body, 0)


@functools.partial(jax.jit, static_argnums=())
def _sc_topk_gather(D, Bm):
    mesh = plsc.VectorSubcoreMesh(core_axis_name="c", subcore_axis_name="s")
    f = pl.kernel(
        _sc_body,
        out_type=jax.ShapeDtypeStruct((_NROWS * _K, 128), jnp.float32),
        mesh=mesh,
        compiler_params=pltpu.CompilerParams(needs_layout_passes=False),
        scratch_types=[
            pltpu.VMEM((_BATCH, _P), jnp.float32),
            pltpu.VMEM((1040,), jnp.int32),
            pltpu.VMEM((64,), jnp.int32),
            pltpu.VMEM((1040,), jnp.int32),
            pltpu.VMEM((128,), jnp.int32),
            pltpu.VMEM((128, 128), jnp.float32),
            pltpu.SemaphoreType.DMA,
        ],
    )
    return f(D, Bm)


def kernel(x, pos, batch, W_c1a, b_c1a, W_c1b, b_c1b, W_c2a, b_c2a,
           W_c2b, b_c2b, W_c3a, b_c3a, W_c3b, b_c3b, W_l1, b_l1,
           W_m1, b_m1, W_m2, b_m2, W_m3, b_m3):
    x0 = jnp.concatenate([x, pos], axis=1).reshape(_B, _P, 9)

    D1, A1, B1 = _tc_dist(x0, W_c1a, b_c1a)
    G1 = _sc_topk_gather(D1.reshape(_NROWS, _P), B1.reshape(_NROWS, 128))
    x1 = _tc_er(G1.reshape(_B, _P * _K, 128), A1, W_c1b, b_c1b)
    D2, A2, B2 = _tc_dist(x1, W_c2a, b_c2a)
    G2 = _sc_topk_gather(D2.reshape(_NROWS, _P), B2.reshape(_NROWS, 128))
    x2 = _tc_er(G2.reshape(_B, _P * _K, 128), A2, W_c2b, b_c2b)
    D3, A3, B3 = _tc_dist(x2, W_c3a, b_c3a)
    G3 = _sc_topk_gather(D3.reshape(_NROWS, _P), B3.reshape(_NROWS, 128))
    x3 = _tc_er(G3.reshape(_B, _P * _K, 128), A3, W_c3b, b_c3b)
    out = _tc_cls(x1, x2, x3, W_l1, b_l1, W_m1, b_m1, W_m2, b_m2, W_m3, b_m3)
    return out.reshape(_B * _P, 40)


# speedup vs baseline: 1.0045x; 1.0045x over previous
import functools

import numpy as np
import jax
import jax.numpy as jnp
from jax import lax
from jax.experimental import pallas as pl
from jax.experimental.pallas import tpu as pltpu
from jax.experimental.pallas import tpu_sc as plsc

_B, _P, _K = 16, 1024, 30
_HI = jax.lax.Precision.HIGHEST

_NROWS = _B * _P
_NW = 32
_RPW = _NROWS // _NW
_BATCH = 4
_NB = _RPW // _BATCH


def _dot(a, w):
    return lax.dot_general(a, w, (((1,), (0,)), ((), ())), precision=_HI)


def _dist_and_tables(x, W1, b1):
    d = x.shape[1]
    d2 = jnp.sum(x * x, axis=1)
    G = lax.dot_general(x, x, (((1,), (1,)), ((), ())), precision=_HI)
    D = d2[:, None] + d2[None, :] - 2.0 * G
    W1a, W1b = W1[:d], W1[d:]
    A = _dot(x, W1a - W1b) + b1
    Bm = _dot(x, W1b)
    Bm = jnp.concatenate([Bm, jnp.zeros((_P, 64), jnp.float32)], axis=1)
    return D, A, Bm


def _edge_reduce(G, A, W2, b2):
    npt = A.shape[0]
    G = G[:, :64]
    h1 = jnp.maximum(A[:, None, :] + G.reshape(npt, _K, 64), 0.0)
    h2 = _dot(h1.reshape(npt * _K, 64), W2) + b2
    h2 = jnp.maximum(h2, 0.0).reshape(npt, _K, 64)
    return jnp.max(h2, axis=1)


def _tc_dist_body(x_ref, W1, b1, d_ref, a_ref, bm_ref):
    D, A, Bm = _dist_and_tables(x_ref[0], W1[...], b1[...])
    d_ref[0] = D
    a_ref[0] = A
    bm_ref[0] = Bm


def _tc_er_body(g_ref, a_ref, W2, b2, x_ref):
    x_ref[0] = _edge_reduce(g_ref[0], a_ref[0], W2[...], b2[...])


def _tc_cls_body(x1_ref, x2_ref, x3_ref, Wl1, bl1, Wm1, bm1,
                 Wm2, bm2, Wm3, bm3, out_ref):
    feat = jnp.concatenate([x1_ref[0], x2_ref[0], x3_ref[0]], axis=1)
    h = jnp.maximum(_dot(feat, Wl1[...]) + bl1[...], 0.0)
    h = jnp.maximum(_dot(h, Wm1[...]) + bm1[...], 0.0)
    h = jnp.maximum(_dot(h, Wm2[...]) + bm2[...], 0.0)
    h = _dot(h, Wm3[...]) + bm3[...]
    m = jnp.max(h, axis=1, keepdims=True)
    s = h - m
    lse = jnp.log(jnp.sum(jnp.exp(s), axis=1, keepdims=True))
    out_ref[0] = s - lse


def _wspec(a):
    nd = a.ndim
    return pl.BlockSpec(a.shape, lambda b, _n=nd: (0,) * _n)


def _bspec(shape):
    nz = len(shape)
    return pl.BlockSpec((1,) + shape, lambda b, _n=nz: (b,) + (0,) * _n)


_PT = 128
_NT = _P // _PT


def _tc_dist(x, W1, b1):
    d = x.shape[2]
    return pl.pallas_call(
        _tc_dist_body,
        grid=(_B,),
        in_specs=[_bspec((_P, d)), _wspec(W1), _wspec(b1)],
        out_specs=[_bspec((_P, _P)), _bspec((_P, 64)), _bspec((_P, 128))],
        out_shape=[jax.ShapeDtypeStruct((_B, _P, _P), jnp.float32),
                   jax.ShapeDtypeStruct((_B, _P, 64), jnp.float32),
                   jax.ShapeDtypeStruct((_B, _P, 128), jnp.float32)],
    )(x, W1, b1)


def _tspec(shape):
    nz = len(shape) - 1
    return pl.BlockSpec((1,) + shape,
                        lambda b, t, _n=nz: (b, t) + (0,) * _n)


def _tc_er(G, A, W2, b2):
    return pl.pallas_call(
        _tc_er_body,
        grid=(_B, _NT),
        in_specs=[_tspec((_PT * _K, 128)), _tspec((_PT, 64)),
                  pl.BlockSpec(W2.shape, lambda b, t: (0, 0)),
                  pl.BlockSpec(b2.shape, lambda b, t: (0,))],
        out_specs=_tspec((_PT, 64)),
        out_shape=jax.ShapeDtypeStruct((_B, _P, 64), jnp.float32),
    )(G, A, W2, b2)


def _tc_cls(x1, x2, x3, Wl1, bl1, Wm1, bm1, Wm2, bm2, Wm3, bm3):
    ws = [Wl1, bl1, Wm1, bm1, Wm2, bm2, Wm3, bm3]
    return pl.pallas_call(
        _tc_cls_body,
        grid=(_B,),
        in_specs=[_bspec((_P, 64))] * 3 + [_wspec(a) for a in ws],
        out_specs=_bspec((_P, 40)),
        out_shape=jax.ShapeDtypeStruct((_B, _P, 40), jnp.float32),
    )(x1, x2, x3, *ws)


def _merge32(L, H, s):
    rs = lax.rev(s, (0,))
    lo = jnp.minimum(L, rs)
    hi = jnp.maximum(L, rs)
    L2 = lax.sort(lo)
    hi_s = lax.sort(hi)
    lo2 = jnp.minimum(hi_s, lax.rev(H, (0,)))
    H2 = lax.sort(lo2)
    return L2, H2


def _pair32(a, b):
    rb = lax.rev(b, (0,))
    lo = jnp.minimum(a, rb)
    hi = jnp.maximum(a, rb)
    return lax.sort(lo), lax.sort(hi)


def _lane(vec, j, iota, neutral):
    return jnp.max(jnp.where(iota == j, vec, neutral))


def _sc_body(d_hbm, bm_hbm, g_hbm, drow2, sidx, pidx, eqbuf, fidx2, gstage2,
             gsem, dsem):
    wid = lax.axis_index("s") * 2 + lax.axis_index("c")
    base = wid * _RPW
    iota = lax.iota(jnp.int32, 16)
    inf16 = jnp.full((16,), np.inf, jnp.float32)
    ninf = jnp.float32(-np.inf)
    imax16 = jnp.full((16,), np.int32(2147483647), jnp.int32)

    for q in range(2):
        for i in range(8):
            fidx2[q, pl.ds(i * 16, 16)] = jnp.zeros((16,), jnp.int32)

    def select_row(p, cbase, drow, fidx):
        pfull = jnp.full((16,), p, jnp.int32)
        acc = [drow[p, pl.ds(k * 16, 16)] for k in range(4)]
        for c in range(4, 64):
            acc[c % 4] = jnp.minimum(acc[c % 4], drow[p, pl.ds(c * 16, 16)])
        s0, s1, s2, s3 = (lax.sort(a) for a in acc)
        La, Ha = _pair32(s0, s1)
        Lb, Hb = _pair32(s2, s3)
        L, H = _merge32(La, Ha, Lb)
        _, H = _merge32(L, H, Hb)
        tb = jnp.full((16,), _lane(H, 13, iota, ninf), jnp.float32)

        def f2(cc, off):
            c0 = cc * 4
            masks = []
            pcs = []
            for u in range(4):
                dv = drow[p, pl.ds((c0 + u) * 16, 16)]
                m_ = dv <= tb
                masks.append(m_)
                pcs.append(jnp.sum(m_.astype(jnp.int32)))
            o = off
            for u in range(4):
                plsc.store_compressed(sidx.at[pl.ds(o, 16)],
                                      iota + (c0 + u) * 16, mask=masks[u])
                o = o + pcs[u]
            return o

        n = lax.fori_loop(0, 16, f2, jnp.int32(0))
        nv = (n + 15) // 16

        def f3(v, LH):
            b16 = v * 16
            valid = (iota + b16) < n
            idxv = jnp.where(valid, sidx[pl.ds(b16, 16)], 0)
            keys = plsc.load_gather(drow, [pfull, idxv])
            keys = jnp.where(valid, keys, inf16)
            return _merge32(*LH, lax.sort(keys))

        _, H = lax.fori_loop(0, nv, f3, (inf16, inf16))
        t30 = jnp.full((16,), _lane(H, 13, iota, ninf), jnp.float32)

        def f4(v, carry):
            clt, neq = carry
            b16 = v * 16
            valid = (iota + b16) < n
            idxv = jnp.where(valid, sidx[pl.ds(b16, 16)], 0)
            keys = plsc.load_gather(drow, [pfull, idxv])
            klt = (keys < t30) & valid
            keq = (keys == t30) & valid
            plsc.store_compressed(pidx.at[pl.ds(clt, 16)], idxv, mask=klt)
            plsc.store_compressed(eqbuf.at[pl.ds(neq, 16)], idxv, mask=keq)
            return (clt + jnp.sum(klt.astype(jnp.int32)),
                    neq + jnp.sum(keq.astype(jnp.int32)))

        clt, neq = lax.fori_loop(0, nv, f4, (jnp.int32(0), jnp.int32(0)))
        need = 30 - clt

        nve = (neq + 15) // 16

        def f5(v, LH):
            b16 = v * 16
            valid = (iota + b16) < neq
            idxv = jnp.where(valid, eqbuf[pl.ds(b16, 16)], 0)
            idxv = jnp.where(valid, idxv, imax16)
            return _merge32(*LH, lax.sort(idxv))

        Li, Hi = lax.fori_loop(0, nve, f5, (imax16, imax16))
        nlo = jnp.minimum(need, 16)
        plsc.store_compressed(pidx.at[pl.ds(clt, 16)], Li, mask=iota < nlo)
        plsc.store_compressed(pidx.at[pl.ds(clt + nlo, 16)], Hi,
                              mask=iota < (need - 16))

        fb = p * 30
        fidx[pl.ds(fb, 16)] = pidx[pl.ds(0, 16)] + cbase
        plsc.store_compressed(fidx.at[pl.ds(fb + 16, 16)],
                              pidx[pl.ds(16, 16)] + cbase, mask=iota < 14)

    pltpu.async_copy(d_hbm.at[pl.ds(base, _BATCH)], drow2.at[0], dsem)

    def batch_body(bi, _):
        pt0 = base + bi * _BATCH
        cbase = (pt0 // _P) * _P
        par = bi % 2
        drow = drow2.at[par]
        fidx = fidx2.at[par]
        pltpu.make_async_copy(d_hbm.at[pl.ds(pt0, _BATCH)], drow,
                              dsem).wait()

        @pl.when(bi + 1 < _NB)
        def _():
            pltpu.async_copy(
                d_hbm.at[pl.ds(pt0 + _BATCH, _BATCH)],
                drow2.at[1 - par], dsem)

        for p in range(_BATCH):
            select_row(p, cbase, drow, fidx)

        @pl.when(bi > 0)
        def _():
            ppar = 1 - par
            pltpu.make_async_copy(bm_hbm.at[fidx2.at[ppar]],
                                  gstage2.at[ppar], gsem).wait()
            pltpu.sync_copy(
                gstage2.at[ppar].at[pl.ds(0, _BATCH * 30)],
                g_hbm.at[pl.ds((pt0 - _BATCH) * 30, _BATCH * 30)])

        pltpu.async_copy(bm_hbm.at[fidx], gstage2.at[par], gsem)
        return 0

    lax.fori_loop(0, _NB, batch_body, 0)
    lpar = (_NB - 1) % 2
    lpt0 = base + (_NB - 1) * _BATCH
    pltpu.make_async_copy(bm_hbm.at[fidx2.at[lpar]], gstage2.at[lpar],
                          gsem).wait()
    pltpu.sync_copy(gstage2.at[lpar].at[pl.ds(0, _BATCH * 30)],
                    g_hbm.at[pl.ds(lpt0 * 30, _BATCH * 30)])


@functools.partial(jax.jit, static_argnums=())
def _sc_topk_gather(D, Bm):
    mesh = plsc.VectorSubcoreMesh(core_axis_name="c", subcore_axis_name="s")
    f = pl.kernel(
        _sc_body,
        out_type=jax.ShapeDtypeStruct((_NROWS * _K, 128), jnp.float32),
        mesh=mesh,
        compiler_params=pltpu.CompilerParams(needs_layout_passes=False),
        scratch_types=[
            pltpu.VMEM((2, _BATCH, _P), jnp.float32),
            pltpu.VMEM((1040,), jnp.int32),
            pltpu.VMEM((64,), jnp.int32),
            pltpu.VMEM((1040,), jnp.int32),
            pltpu.VMEM((2, 128), jnp.int32),
            pltpu.VMEM((2, 128, 128), jnp.float32),
            pltpu.SemaphoreType.DMA,
            pltpu.SemaphoreType.DMA,
        ],
    )
    return f(D, Bm)


def kernel(x, pos, batch, W_c1a, b_c1a, W_c1b, b_c1b, W_c2a, b_c2a,
           W_c2b, b_c2b, W_c3a, b_c3a, W_c3b, b_c3b, W_l1, b_l1,
           W_m1, b_m1, W_m2, b_m2, W_m3, b_m3):
    x0 = jnp.concatenate([x, pos], axis=1).reshape(_B, _P, 9)

    D1, A1, B1 = _tc_dist(x0, W_c1a, b_c1a)
    G1 = _sc_topk_gather(D1.reshape(_NROWS, _P), B1.reshape(_NROWS, 128))
    x1 = _tc_er(G1.reshape(_B, _P * _K, 128), A1, W_c1b, b_c1b)
    D2, A2, B2 = _tc_dist(x1, W_c2a, b_c2a)
    G2 = _sc_topk_gather(D2.reshape(_NROWS, _P), B2.reshape(_NROWS, 128))
    x2 = _tc_er(G2.reshape(_B, _P * _K, 128), A2, W_c2b, b_c2b)
    D3, A3, B3 = _tc_dist(x2, W_c3a, b_c3a)
    G3 = _sc_topk_gather(D3.reshape(_NROWS, _P), B3.reshape(_NROWS, 128))
    x3 = _tc_er(G3.reshape(_B, _P * _K, 128), A3, W_c3b, b_c3b)
    out = _tc_cls(x1, x2, x3, W_l1, b_l1, W_m1, b_m1, W_m2, b_m2, W_m3, b_m3)
    return out.reshape(_B * _P, 40)
